# baseline (speedup 1.0000x reference)
import jax
import jax.numpy as jnp
from jax.experimental import pallas as pl
from jax.experimental.pallas import tpu as pltpu

_NUM_SYM = 14
_FFT = 4096
_N = 2
_ROW_IN = 12 * _FFT * _N
_ROW_OUT = _NUM_SYM * _FFT * _N
_SEG = _FFT * _N


def _body(x_hbm, p_ref, o_hbm, pil_ref, sems):
    b = x_hbm.shape[0]
    pr = jnp.repeat(p_ref[...], _N, axis=1)
    pil_ref[0] = jnp.broadcast_to(pr[0:1, :], (b, _SEG))
    pil_ref[1] = jnp.broadcast_to(pr[1:2, :], (b, _SEG))
    cps = [
        pltpu.make_async_copy(x_hbm.at[:, 0:2 * _SEG],
                              o_hbm.at[:, 0:2 * _SEG], sems.at[0]),
        pltpu.make_async_copy(x_hbm.at[:, 2 * _SEG:10 * _SEG],
                              o_hbm.at[:, 3 * _SEG:11 * _SEG], sems.at[1]),
        pltpu.make_async_copy(x_hbm.at[:, 10 * _SEG:12 * _SEG],
                              o_hbm.at[:, 12 * _SEG:14 * _SEG], sems.at[2]),
        pltpu.make_async_copy(pil_ref.at[0],
                              o_hbm.at[:, 2 * _SEG:3 * _SEG], sems.at[3]),
        pltpu.make_async_copy(pil_ref.at[1],
                              o_hbm.at[:, 11 * _SEG:12 * _SEG], sems.at[4]),
    ]
    for c in cps:
        c.start()
    for c in cps:
        c.wait()


def kernel(inputs, pilots):
    b = inputs.shape[0]
    x = inputs.reshape(b, _ROW_IN)
    p2 = pilots.reshape(2, _FFT)
    out = pl.pallas_call(
        _body,
        in_specs=[
            pl.BlockSpec(memory_space=pltpu.MemorySpace.HBM),
            pl.BlockSpec(memory_space=pltpu.MemorySpace.VMEM),
        ],
        out_specs=pl.BlockSpec(memory_space=pltpu.MemorySpace.HBM),
        out_shape=jax.ShapeDtypeStruct((b, _ROW_OUT), inputs.dtype),
        scratch_shapes=[
            pltpu.VMEM((2, b, _SEG), jnp.float32),
            pltpu.SemaphoreType.DMA((5,)),
        ],
    )(x, p2)
    return out.reshape(b, 1, 1, _NUM_SYM, _FFT, _N)


# speedup vs baseline: 7.0361x; 7.0361x over previous
import jax
import jax.numpy as jnp
from jax.experimental import pallas as pl

_NUM_SYM = 14
_FFT = 4096
_N = 2
_BATCH = 128
_ROW_IN = 12 * _FFT * _N
_ROW_OUT = _NUM_SYM * _FFT * _N
_SEG = _FFT * _N

_BB = 16


def _body(x_ref, p_ref, o_ref):
    o_ref[:, 0:2 * _SEG] = x_ref[:, 0:2 * _SEG]
    o_ref[:, 3 * _SEG:11 * _SEG] = x_ref[:, 2 * _SEG:10 * _SEG]
    o_ref[:, 12 * _SEG:14 * _SEG] = x_ref[:, 10 * _SEG:12 * _SEG]
    pr = jnp.repeat(p_ref[...], _N, axis=1)
    o_ref[:, 2 * _SEG:3 * _SEG] = jnp.broadcast_to(pr[0:1, :], (_BB, _SEG))
    o_ref[:, 11 * _SEG:12 * _SEG] = jnp.broadcast_to(pr[1:2, :], (_BB, _SEG))


def kernel(inputs, pilots):
    b = inputs.shape[0]
    x = inputs.reshape(b, _ROW_IN)
    p2 = pilots.reshape(2, _FFT)
    out = pl.pallas_call(
        _body,
        grid=(b // _BB,),
        in_specs=[
            pl.BlockSpec((_BB, _ROW_IN), lambda i: (i, 0)),
            pl.BlockSpec((2, _FFT), lambda i: (0, 0)),
        ],
        out_specs=pl.BlockSpec((_BB, _ROW_OUT), lambda i: (i, 0)),
        out_shape=jax.ShapeDtypeStruct((b, _ROW_OUT), inputs.dtype),
    )(x, p2)
    return out.reshape(b, 1, 1, _NUM_SYM, _FFT, _N)


# speedup vs baseline: 7.3591x; 1.0459x over previous
import jax
import jax.numpy as jnp
from jax.experimental import pallas as pl

_NUM_SYM = 14
_FFT = 4096
_N = 2
_BATCH = 128
_ROW_IN = 12 * _FFT * _N
_ROW_OUT = _NUM_SYM * _FFT * _N
_SEG = _FFT * _N

_BB = 32


def _body(x_ref, p_ref, o_ref):
    o_ref[:, 0:2 * _SEG] = x_ref[:, 0:2 * _SEG]
    o_ref[:, 3 * _SEG:11 * _SEG] = x_ref[:, 2 * _SEG:10 * _SEG]
    o_ref[:, 12 * _SEG:14 * _SEG] = x_ref[:, 10 * _SEG:12 * _SEG]
    pr = jnp.repeat(p_ref[...], _N, axis=1)
    o_ref[:, 2 * _SEG:3 * _SEG] = jnp.broadcast_to(pr[0:1, :], (_BB, _SEG))
    o_ref[:, 11 * _SEG:12 * _SEG] = jnp.broadcast_to(pr[1:2, :], (_BB, _SEG))


def kernel(inputs, pilots):
    b = inputs.shape[0]
    x = inputs.reshape(b, _ROW_IN)
    p2 = pilots.reshape(2, _FFT)
    out = pl.pallas_call(
        _body,
        grid=(b // _BB,),
        in_specs=[
            pl.BlockSpec((_BB, _ROW_IN), lambda i: (i, 0)),
            pl.BlockSpec((2, _FFT), lambda i: (0, 0)),
        ],
        out_specs=pl.BlockSpec((_BB, _ROW_OUT), lambda i: (i, 0)),
        out_shape=jax.ShapeDtypeStruct((b, _ROW_OUT), inputs.dtype),
    )(x, p2)
    return out.reshape(b, 1, 1, _NUM_SYM, _FFT, _N)


# speedup vs baseline: 10.4082x; 1.4143x over previous
import jax
import jax.numpy as jnp
from jax.experimental import pallas as pl

_NUM_SYM = 14
_FFT = 4096
_N = 2
_BATCH = 128
_TIN = 384
_TOUT = 448
_TS = 32

_BB = 32


def _body(x_ref, p_ref, o_ref):
    o_ref[:, 0:2 * _TS] = x_ref[:, 0:2 * _TS]
    o_ref[:, 3 * _TS:11 * _TS] = x_ref[:, 2 * _TS:10 * _TS]
    o_ref[:, 12 * _TS:14 * _TS] = x_ref[:, 10 * _TS:12 * _TS]
    p = p_ref[...]
    p0 = jnp.concatenate([p[0:_TS], p[0:_TS]], axis=1)
    p1 = jnp.concatenate([p[_TS:2 * _TS], p[_TS:2 * _TS]], axis=1)
    o_ref[:, 2 * _TS:3 * _TS] = jnp.broadcast_to(p0[None], (_BB, _TS, 256))
    o_ref[:, 11 * _TS:12 * _TS] = jnp.broadcast_to(p1[None], (_BB, _TS, 256))


def kernel(inputs, pilots):
    b = inputs.shape[0]
    x = inputs.reshape(b, _TIN, 128, _N).transpose(0, 1, 3, 2).reshape(b, _TIN, _N * 128)
    p = pilots.reshape(64, 128)
    out = pl.pallas_call(
        _body,
        grid=(b // _BB,),
        in_specs=[
            pl.BlockSpec((_BB, _TIN, _N * 128), lambda i: (i, 0, 0)),
            pl.BlockSpec((64, 128), lambda i: (0, 0)),
        ],
        out_specs=pl.BlockSpec((_BB, _TOUT, _N * 128), lambda i: (i, 0, 0)),
        out_shape=jax.ShapeDtypeStruct((b, _TOUT, _N * 128), inputs.dtype),
    )(x, p)
    return (out.reshape(b, _TOUT, _N, 128)
               .transpose(0, 1, 3, 2)
               .reshape(b, 1, 1, _NUM_SYM, _FFT, _N))
